# Initial kernel scaffold; baseline (speedup 1.0000x reference)
import jax
import jax.numpy as jnp
from jax.experimental import pallas as pl


def kernel(hidden_state, W_in, b_in, W_out, b_out, codebooks):
    raise NotImplementedError("write your pallas kernel here")


# speedup vs baseline: 3.5127x; 3.5127x over previous
import functools

import jax
import jax.numpy as jnp
from jax import lax
from jax.experimental import pallas as pl


def _rvq_body(NQ, K, C, D, TS, n_tb,
              x_ref, win_ref, bin_ref, wout_ref, bout_ref, cb_ref,
              q_ref, idx_ref, plat_ref, loss_ref):
    g = pl.program_id(0)
    r = x_ref[0]
    qtot = jnp.zeros((D, TS), jnp.float32)
    loss = jnp.float32(0.0)
    idx_rows = []
    proj_rows = []
    for i in range(NQ):
        Wi = win_ref[i]
        proj = jnp.dot(Wi, r, preferred_element_type=jnp.float32)
        proj = proj + bin_ref[i][:, None]
        nrm = jnp.sqrt(jnp.sum(proj * proj, axis=0, keepdims=True))
        enc_n = proj / jnp.maximum(nrm, 1e-12)
        l2 = jnp.sum(enc_n * enc_n, axis=0, keepdims=True)
        cb = cb_ref[i]
        cbn = cb / jnp.maximum(
            jnp.sqrt(jnp.sum(cb * cb, axis=1, keepdims=True)), 1e-12)
        cb2 = jnp.sum(cbn * cbn, axis=1, keepdims=True)
        scores = jnp.dot(cbn, enc_n, preferred_element_type=jnp.float32)
        dist = l2 - 2.0 * scores + cb2
        idx = jnp.argmax(-dist, axis=0)
        onehot = (lax.broadcasted_iota(jnp.int32, (K, TS), 0)
                  == idx[None, :]).astype(jnp.float32)
        quant = lax.dot_general(cb, onehot, (((0,), (0,)), ((), ())),
                                preferred_element_type=jnp.float32)
        diff = proj - quant
        loss = loss + jnp.sum(diff * diff)
        qo = jnp.dot(wout_ref[i], quant, preferred_element_type=jnp.float32)
        qo = qo + bout_ref[i][:, None]
        qtot = qtot + qo
        r = r - qo
        idx_rows.append(idx[None, :].astype(jnp.int32))
        proj_rows.append(proj)
    q_ref[0] = qtot
    pad = 16 - NQ
    idx_ref[0] = jnp.concatenate(
        idx_rows + [jnp.zeros((pad, TS), jnp.int32)], axis=0)
    plat_ref[0] = jnp.concatenate(proj_rows, axis=0)

    @pl.when(g == 0)
    def _():
        loss_ref[:, :] = jnp.zeros((1, 1), jnp.float32)

    loss_ref[:, :] += jnp.reshape(loss, (1, 1))


def kernel(hidden_state, W_in, b_in, W_out, b_out, codebooks):
    B, D, T = hidden_state.shape
    NQ, C, _ = W_in.shape
    K = codebooks.shape[1]
    TS = 512 if T % 512 == 0 else T
    n_tb = T // TS
    grid = (B * n_tb,)

    body = functools.partial(_rvq_body, NQ, K, C, D, TS, n_tb)
    out_shape = [
        jax.ShapeDtypeStruct((B, D, T), jnp.float32),
        jax.ShapeDtypeStruct((B, 16, T), jnp.int32),
        jax.ShapeDtypeStruct((B, NQ * C, T), jnp.float32),
        jax.ShapeDtypeStruct((1, 1), jnp.float32),
    ]
    in_specs = [
        pl.BlockSpec((1, D, TS), lambda g: (g // n_tb, 0, g % n_tb)),
        pl.BlockSpec((NQ, C, D), lambda g: (0, 0, 0)),
        pl.BlockSpec((NQ, C), lambda g: (0, 0)),
        pl.BlockSpec((NQ, D, C), lambda g: (0, 0, 0)),
        pl.BlockSpec((NQ, D), lambda g: (0, 0)),
        pl.BlockSpec((NQ, K, C), lambda g: (0, 0, 0)),
    ]
    out_specs = [
        pl.BlockSpec((1, D, TS), lambda g: (g // n_tb, 0, g % n_tb)),
        pl.BlockSpec((1, 16, TS), lambda g: (g // n_tb, 0, g % n_tb)),
        pl.BlockSpec((1, NQ * C, TS), lambda g: (g // n_tb, 0, g % n_tb)),
        pl.BlockSpec((1, 1), lambda g: (0, 0)),
    ]
    quantized, idx_pad, proj_lat, loss_sum = pl.pallas_call(
        body,
        grid=grid,
        in_specs=in_specs,
        out_specs=out_specs,
        out_shape=out_shape,
    )(hidden_state, W_in, b_in, W_out, b_out, codebooks)

    indices = idx_pad[:, :NQ, :]
    total = loss_sum[0, 0] * (1.0 / (B * C * T))
    return (quantized, indices, proj_lat, total, total)


# speedup vs baseline: 4.0343x; 1.1485x over previous
import functools

import jax
import jax.numpy as jnp
from jax import lax
from jax.experimental import pallas as pl
from jax.experimental.pallas import tpu as pltpu


def _rvq_body(NQ, K, C, D, TS, n_tb,
              x_ref, win_ref, bin_ref, wout_ref, bout_ref, cb_ref,
              q_ref, idx_ref, plat_ref, loss_ref):
    r = x_ref[0]
    qtot = jnp.zeros((D, TS), jnp.float32)
    loss = jnp.float32(0.0)
    idx_rows = []
    proj_rows = []
    kiota = lax.broadcasted_iota(jnp.int32, (K, TS), 0)
    for i in range(NQ):
        Wi = win_ref[i]
        proj = jnp.dot(Wi, r, preferred_element_type=jnp.float32)
        proj = proj + bin_ref[i][:, None]
        nrm = jnp.sqrt(jnp.sum(proj * proj, axis=0, keepdims=True))
        enc_n = proj / jnp.maximum(nrm, 1e-12)
        l2 = jnp.sum(enc_n * enc_n, axis=0, keepdims=True)
        cb = cb_ref[i]
        cbn = cb / jnp.maximum(
            jnp.sqrt(jnp.sum(cb * cb, axis=1, keepdims=True)), 1e-12)
        cb2 = jnp.sum(cbn * cbn, axis=1, keepdims=True)
        scores = jnp.dot(cbn, enc_n, preferred_element_type=jnp.float32)
        dist = l2 - 2.0 * scores + cb2
        dmin = jnp.min(dist, axis=0, keepdims=True)
        idx = jnp.min(jnp.where(dist == dmin, kiota, K), axis=0)
        onehot = (kiota == idx[None, :]).astype(jnp.float32)
        quant = lax.dot_general(cb, onehot, (((0,), (0,)), ((), ())),
                                preferred_element_type=jnp.float32)
        diff = proj - quant
        loss = loss + jnp.sum(diff * diff)
        qo = jnp.dot(wout_ref[i], quant, preferred_element_type=jnp.float32)
        qo = qo + bout_ref[i][:, None]
        qtot = qtot + qo
        r = r - qo
        idx_rows.append(idx[None, :].astype(jnp.int32))
        proj_rows.append(proj)
    q_ref[0] = qtot
    pad = 16 - NQ
    idx_ref[0] = jnp.concatenate(
        idx_rows + [jnp.zeros((pad, TS), jnp.int32)], axis=0)
    plat_ref[0] = jnp.concatenate(proj_rows, axis=0)
    loss_ref[:, :, :] = jnp.reshape(loss, (1, 1, 1))


def kernel(hidden_state, W_in, b_in, W_out, b_out, codebooks):
    B, D, T = hidden_state.shape
    NQ, C, _ = W_in.shape
    K = codebooks.shape[1]
    TS = 1024 if T % 1024 == 0 else T
    n_tb = T // TS
    grid = (B * n_tb,)

    body = functools.partial(_rvq_body, NQ, K, C, D, TS, n_tb)
    out_shape = [
        jax.ShapeDtypeStruct((B, D, T), jnp.float32),
        jax.ShapeDtypeStruct((B, 16, T), jnp.int32),
        jax.ShapeDtypeStruct((B, NQ * C, T), jnp.float32),
        jax.ShapeDtypeStruct((B * n_tb, 1, 1), jnp.float32),
    ]
    in_specs = [
        pl.BlockSpec((1, D, TS), lambda g: (g // n_tb, 0, g % n_tb)),
        pl.BlockSpec((NQ, C, D), lambda g: (0, 0, 0)),
        pl.BlockSpec((NQ, C), lambda g: (0, 0)),
        pl.BlockSpec((NQ, D, C), lambda g: (0, 0, 0)),
        pl.BlockSpec((NQ, D), lambda g: (0, 0)),
        pl.BlockSpec((NQ, K, C), lambda g: (0, 0, 0)),
    ]
    out_specs = [
        pl.BlockSpec((1, D, TS), lambda g: (g // n_tb, 0, g % n_tb)),
        pl.BlockSpec((1, 16, TS), lambda g: (g // n_tb, 0, g % n_tb)),
        pl.BlockSpec((1, NQ * C, TS), lambda g: (g // n_tb, 0, g % n_tb)),
        pl.BlockSpec((1, 1, 1), lambda g: (g, 0, 0)),
    ]
    quantized, idx_pad, proj_lat, loss_part = pl.pallas_call(
        body,
        grid=grid,
        in_specs=in_specs,
        out_specs=out_specs,
        out_shape=out_shape,
        compiler_params=pltpu.CompilerParams(
            dimension_semantics=("parallel",)),
    )(hidden_state, W_in, b_in, W_out, b_out, codebooks)

    indices = idx_pad[:, :NQ, :]
    total = jnp.sum(loss_part) * (1.0 / (B * C * T))
    return (quantized, indices, proj_lat, total, total)


# speedup vs baseline: 4.2178x; 1.0455x over previous
import functools

import jax
import jax.numpy as jnp
from jax import lax
from jax.experimental import pallas as pl
from jax.experimental.pallas import tpu as pltpu


def _rvq_body(NQ, K, C, D, TS, n_tb,
              x_ref, win_ref, bin_ref, wout_ref, bout_ref, cb_ref,
              q_ref, idx_ref, plat_ref, loss_ref):
    r0 = x_ref[0]
    r = r0
    loss = jnp.float32(0.0)
    idx_rows = []
    proj_rows = []
    kiota = lax.broadcasted_iota(jnp.int32, (K, TS), 0)
    for i in range(NQ):
        Wi = win_ref[i]
        proj = jnp.dot(Wi, r, preferred_element_type=jnp.float32)
        proj = proj + bin_ref[i][:, None]
        nrm = jnp.sqrt(jnp.sum(proj * proj, axis=0, keepdims=True))
        enc_n = proj / jnp.maximum(nrm, 1e-12)
        l2 = jnp.sum(enc_n * enc_n, axis=0, keepdims=True)
        cb = cb_ref[i]
        cbn = cb / jnp.maximum(
            jnp.sqrt(jnp.sum(cb * cb, axis=1, keepdims=True)), 1e-12)
        cb2 = jnp.sum(cbn * cbn, axis=1, keepdims=True)
        scores = jnp.dot(cbn, enc_n, preferred_element_type=jnp.float32)
        dist = l2 - 2.0 * scores + cb2
        dmin = jnp.min(dist, axis=0, keepdims=True)
        idx = jnp.min(jnp.where(dist == dmin, kiota, K), axis=0)
        onehot = (kiota == idx[None, :]).astype(jnp.float32)
        quant = lax.dot_general(cb, onehot, (((0,), (0,)), ((), ())),
                                preferred_element_type=jnp.float32)
        diff = proj - quant
        loss = loss + jnp.sum(diff * diff)
        qo = jnp.dot(wout_ref[i], quant, preferred_element_type=jnp.float32)
        qo = qo + bout_ref[i][:, None]
        r = r - qo
        idx_rows.append(idx[None, :].astype(jnp.int32))
        proj_rows.append(proj)
    q_ref[0] = r0 - r
    pad = 16 - NQ
    idx_ref[0] = jnp.concatenate(
        idx_rows + [jnp.zeros((pad, TS), jnp.int32)], axis=0)
    plat_ref[0] = jnp.concatenate(proj_rows, axis=0)
    loss_ref[:, :, :] = jnp.reshape(loss, (1, 1, 1))


def kernel(hidden_state, W_in, b_in, W_out, b_out, codebooks):
    B, D, T = hidden_state.shape
    NQ, C, _ = W_in.shape
    K = codebooks.shape[1]
    TS = 1024 if T % 1024 == 0 else T
    n_tb = T // TS
    grid = (B * n_tb,)

    body = functools.partial(_rvq_body, NQ, K, C, D, TS, n_tb)
    out_shape = [
        jax.ShapeDtypeStruct((B, D, T), jnp.float32),
        jax.ShapeDtypeStruct((B, 16, T), jnp.int32),
        jax.ShapeDtypeStruct((B, NQ * C, T), jnp.float32),
        jax.ShapeDtypeStruct((B * n_tb, 1, 1), jnp.float32),
    ]
    in_specs = [
        pl.BlockSpec((1, D, TS), lambda g: (g // n_tb, 0, g % n_tb)),
        pl.BlockSpec((NQ, C, D), lambda g: (0, 0, 0)),
        pl.BlockSpec((NQ, C), lambda g: (0, 0)),
        pl.BlockSpec((NQ, D, C), lambda g: (0, 0, 0)),
        pl.BlockSpec((NQ, D), lambda g: (0, 0)),
        pl.BlockSpec((NQ, K, C), lambda g: (0, 0, 0)),
    ]
    out_specs = [
        pl.BlockSpec((1, D, TS), lambda g: (g // n_tb, 0, g % n_tb)),
        pl.BlockSpec((1, 16, TS), lambda g: (g // n_tb, 0, g % n_tb)),
        pl.BlockSpec((1, NQ * C, TS), lambda g: (g // n_tb, 0, g % n_tb)),
        pl.BlockSpec((1, 1, 1), lambda g: (g, 0, 0)),
    ]
    quantized, idx_pad, proj_lat, loss_part = pl.pallas_call(
        body,
        grid=grid,
        in_specs=in_specs,
        out_specs=out_specs,
        out_shape=out_shape,
        compiler_params=pltpu.CompilerParams(
            dimension_semantics=("parallel",)),
    )(hidden_state, W_in, b_in, W_out, b_out, codebooks)

    indices = idx_pad[:, :NQ, :]
    total = jnp.sum(loss_part) * (1.0 / (B * C * T))
    return (quantized, indices, proj_lat, total, total)
